# baseline (speedup 1.0000x reference)
import jax
import jax.numpy as jnp
from jax import lax
from jax.experimental import pallas as pl

_NUM_EXPERTS = 64
_HIDDEN = 1024
_TOP_K = 2
_BT = 1024


def _router_body(x_ref, w_ref, wout_ref, logits_ref, iout_ref):
    step = pl.program_id(0)
    x = x_ref[...]
    w = w_ref[...]
    logits = lax.dot_general(
        x, w, (((1,), (1,)), ((), ())), preferred_element_type=jnp.float32
    )
    logits_ref[...] = logits

    lt = logits.T
    e_iota = lax.broadcasted_iota(jnp.int32, lt.shape, 0)
    m1 = jnp.max(lt, axis=0, keepdims=True)
    i1 = jnp.min(jnp.where(lt == m1, e_iota, _NUM_EXPERTS), axis=0,
                 keepdims=True)
    masked = jnp.where(e_iota == i1, -jnp.inf, lt)
    m2 = jnp.max(masked, axis=0, keepdims=True)
    i2 = jnp.min(jnp.where(masked == m2, e_iota, _NUM_EXPERTS), axis=0,
                 keepdims=True)

    e2 = jnp.exp(m2 - m1)
    s = 1.0 + e2
    w1 = 1.0 / s
    w2 = e2 / s
    cols = pl.ds(step * _BT, _BT)
    wout_ref[:, cols] = jnp.concatenate([w1, w2], axis=0)
    iout_ref[:, cols] = jnp.concatenate([i1, i2], axis=0)


def kernel(hidden_states, W):
    T, H = hidden_states.shape
    E = W.shape[0]
    grid = (T // _BT,)
    wout, logits, iout = pl.pallas_call(
        _router_body,
        grid=grid,
        in_specs=[
            pl.BlockSpec((_BT, H), lambda i: (i, 0)),
            pl.BlockSpec((E, H), lambda i: (0, 0)),
        ],
        out_specs=[
            pl.BlockSpec((_TOP_K, T), lambda i: (0, 0)),
            pl.BlockSpec((_BT, E), lambda i: (i, 0)),
            pl.BlockSpec((_TOP_K, T), lambda i: (0, 0)),
        ],
        out_shape=[
            jax.ShapeDtypeStruct((_TOP_K, T), jnp.float32),
            jax.ShapeDtypeStruct((T, E), jnp.float32),
            jax.ShapeDtypeStruct((_TOP_K, T), jnp.int32),
        ],
    )(hidden_states, W)
    return (wout.T, logits, iout.T)


# speedup vs baseline: 1.1372x; 1.1372x over previous
import jax
import jax.numpy as jnp
from jax import lax
from jax.experimental import pallas as pl

_NUM_EXPERTS = 64
_HIDDEN = 1024
_TOP_K = 2
_BT = 2048


def _router_body(x_ref, w_ref, wout_ref, logits_ref, iout_ref):
    step = pl.program_id(0)
    x = x_ref[...]
    w = w_ref[...]
    logits = lax.dot_general(
        x, w, (((1,), (1,)), ((), ())), preferred_element_type=jnp.float32
    )
    logits_ref[...] = logits

    lt = logits.T
    e_iota = lax.broadcasted_iota(jnp.int32, lt.shape, 0)
    m1 = jnp.max(lt, axis=0, keepdims=True)
    i1 = jnp.min(jnp.where(lt == m1, e_iota, _NUM_EXPERTS), axis=0,
                 keepdims=True)
    masked = jnp.where(e_iota == i1, -jnp.inf, lt)
    m2 = jnp.max(masked, axis=0, keepdims=True)
    i2 = jnp.min(jnp.where(masked == m2, e_iota, _NUM_EXPERTS), axis=0,
                 keepdims=True)

    e2 = jnp.exp(m2 - m1)
    s = 1.0 + e2
    w1 = 1.0 / s
    w2 = e2 / s
    cols = pl.ds(step * _BT, _BT)
    wout_ref[:, cols] = jnp.concatenate([w1, w2], axis=0)
    iout_ref[:, cols] = jnp.concatenate([i1, i2], axis=0)


def kernel(hidden_states, W):
    T, H = hidden_states.shape
    E = W.shape[0]
    grid = (T // _BT,)
    wout, logits, iout = pl.pallas_call(
        _router_body,
        grid=grid,
        in_specs=[
            pl.BlockSpec((_BT, H), lambda i: (i, 0)),
            pl.BlockSpec((E, H), lambda i: (0, 0)),
        ],
        out_specs=[
            pl.BlockSpec((_TOP_K, T), lambda i: (0, 0)),
            pl.BlockSpec((_BT, E), lambda i: (i, 0)),
            pl.BlockSpec((_TOP_K, T), lambda i: (0, 0)),
        ],
        out_shape=[
            jax.ShapeDtypeStruct((_TOP_K, T), jnp.float32),
            jax.ShapeDtypeStruct((T, E), jnp.float32),
            jax.ShapeDtypeStruct((_TOP_K, T), jnp.int32),
        ],
    )(hidden_states, W)
    return (wout.T, logits, iout.T)
